# speedup vs baseline: 1.6709x; 1.6709x over previous
import functools

import jax
import jax.numpy as jnp
from jax import lax
from jax.experimental import pallas as pl
from jax.experimental.pallas import tpu as pltpu
from jax.experimental.pallas import tpu_sc as plsc

D_VOCAB = 100000
D_MODEL = 1024
BATCH = 4
SEQ = 4096

NC = 2
NS = 16
NW = NC * NS

B = BATCH * SEQ
B_PER_W = B // NW
K = 16
NBUF = 4
N_CHUNKS = B_PER_W // K
N_WAVES = N_CHUNKS // NBUF


def _embed_body(idx_hbm, tbl_hbm, out_hbm, idx_v, buf, in_sem, out_sem):
  cid = lax.axis_index("c")
  sid = lax.axis_index("s")
  wid = sid * NC + cid
  base = wid * B_PER_W

  pltpu.sync_copy(idx_hbm.at[wid], idx_v)

  def start_gather(c, slot):
    pltpu.async_copy(tbl_hbm.at[idx_v.at[c]], buf.at[slot], in_sem.at[slot])

  for b in range(NBUF):
    start_gather(b, b)

  def wave(g, issue_next):
    for b in range(NBUF):
      c = g * NBUF + b
      pltpu.make_async_copy(tbl_hbm.at[idx_v.at[c]], buf.at[b],
                            in_sem.at[b]).wait()
      dst = out_hbm.at[pl.ds(base + c * K, K)]
      pltpu.async_copy(buf.at[b], dst, out_sem.at[b])
      pltpu.make_async_copy(buf.at[b], dst, out_sem.at[b]).wait()
      if issue_next:
        start_gather(c + NBUF, b)

  lax.fori_loop(0, N_WAVES - 1, lambda g, _: (wave(g, True), 0)[1], 0)
  wave(N_WAVES - 1, False)


@jax.jit
def _embed(tokens_idx, embed):
  mesh = plsc.VectorSubcoreMesh(core_axis_name="c", subcore_axis_name="s")
  run = pl.kernel(
      _embed_body,
      out_type=jax.ShapeDtypeStruct((B, D_MODEL), jnp.float32),
      mesh=mesh,
      scratch_types=[
          pltpu.VMEM((N_CHUNKS, K), jnp.int32),
          pltpu.VMEM((NBUF, K, D_MODEL), jnp.float32),
          pltpu.SemaphoreType.DMA((NBUF,)),
          pltpu.SemaphoreType.DMA((NBUF,)),
      ],
  )
  return run(tokens_idx, embed)


def kernel(tokens, embed):
  idx = tokens.reshape(NW, N_CHUNKS, K).astype(jnp.int32)
  out = _embed(idx, embed)
  return out.reshape(BATCH, SEQ, D_MODEL)
